# baseline (speedup 1.0000x reference)
import functools

import jax
import jax.numpy as jnp
from jax import lax
from jax.experimental import pallas as pl
from jax.experimental.pallas import tpu as pltpu
from jax.experimental.pallas import tpu_sc as plsc

BATCH = 4096
FIELDS = 26
EPS = 1e-5

_NC = 2
_NS = 16
_NW = _NC * _NS
_N_PER_W = BATCH * FIELDS // _NW
_CHUNK = 128
_NCHUNK = _N_PER_W // _CHUNK


def _gather_body(x_hbm, table_hbm, out_hbm, xv, embv, sem):
    wid = lax.axis_index("s") * _NC + lax.axis_index("c")
    base = wid * _N_PER_W
    pltpu.sync_copy(x_hbm.at[pl.ds(base, _N_PER_W)], xv)

    def fire(j, _):
        pltpu.async_copy(
            table_hbm.at[xv.at[pl.ds(j * _CHUNK, _CHUNK)]],
            embv.at[pl.ds(j * _CHUNK, _CHUNK)],
            sem,
        )
        return _

    lax.fori_loop(0, _NCHUNK, fire, 0)

    def drain(j, _):
        pltpu.make_async_copy(
            table_hbm.at[xv.at[pl.ds(j * _CHUNK, _CHUNK)]],
            embv.at[pl.ds(j * _CHUNK, _CHUNK)],
            sem,
        ).wait()
        return _

    lax.fori_loop(0, _NCHUNK, drain, 0)
    pltpu.sync_copy(embv, out_hbm.at[pl.ds(base, _N_PER_W)])


_gather = functools.partial(
    pl.kernel,
    mesh=plsc.VectorSubcoreMesh(core_axis_name="c", subcore_axis_name="s"),
    out_type=jax.ShapeDtypeStruct((BATCH * FIELDS,), jnp.float32),
    scratch_types=[
        pltpu.VMEM((_N_PER_W,), jnp.int32),
        pltpu.VMEM((_N_PER_W,), jnp.float32),
        pltpu.SemaphoreType.DMA,
    ],
)(_gather_body)


def _finish_body(emb_ref, gamma_ref, beta_ref, out_ref):
    emb = emb_ref[...]
    gamma = gamma_ref[...]
    beta = beta_ref[...]
    mean = jnp.mean(emb, axis=0, keepdims=True)
    var = jnp.mean((emb - mean) * (emb - mean), axis=0, keepdims=True)
    w = gamma * lax.rsqrt(var + EPS)
    c = jnp.sum(beta - w * mean)
    z = jnp.sum(emb * w, axis=1, keepdims=True) + c
    out_ref[...] = 1.0 / (1.0 + jnp.exp(-z))


def _finish(emb, gamma, beta):
    return pl.pallas_call(
        _finish_body,
        out_shape=jax.ShapeDtypeStruct((BATCH, 1), jnp.float32),
    )(emb, gamma, beta)


@jax.jit
def kernel(x, table, gamma, beta):
    x_flat = x.reshape(-1).astype(jnp.int32)
    table_flat = table.reshape(-1)
    emb = _gather(x_flat, table_flat).reshape(BATCH, FIELDS)
    return _finish(emb, gamma.reshape(1, FIELDS), beta.reshape(1, FIELDS))
